# Initial kernel scaffold; baseline (speedup 1.0000x reference)
import jax
import jax.numpy as jnp
from jax.experimental import pallas as pl


def kernel(xyz, features, W_lse1, W_lse2, W_s1a, W_s1b, b_s1b, W_m1, W_s2a, W_s2b, b_s2b, W_m2, W_d1, W_d2):
    raise NotImplementedError("write your pallas kernel here")


# speedup vs baseline: 1.0016x; 1.0016x over previous
import jax
import jax.numpy as jnp
from jax.experimental import pallas as pl
from jax.experimental.pallas import tpu as pltpu

B, N, C = 4, 4096, 64
HALF, OUT, K = 64, 128, 16


def _bn(x, eps=1e-5):
    axes = tuple(range(x.ndim - 1))
    m = jnp.mean(x, axis=axes, keepdims=True)
    v = jnp.var(x, axis=axes, keepdims=True)
    return (x - m) / jnp.sqrt(v + eps)


def _gather(a, idx):
    return jax.vmap(lambda t, i: t[i])(a, idx)


def _ap(x, Wa, Wb, bb, Wm):
    s = jnp.einsum('bnkc,oc->bnko', x, Wa)
    s = jax.nn.relu(_bn(s))
    s = jnp.einsum('bnkc,oc->bnko', s, Wb) + bb
    s = jax.nn.softmax(s, axis=2)
    f = jnp.sum(x * s, axis=2)
    f = jnp.einsum('bnc,oc->bno', f, Wm)
    return jax.nn.relu(_bn(f))


def _drb_body(agg_ref, w1_ref, w2_ref, out_ref):
    agg = agg_ref[...]
    h = jnp.dot(agg, w1_ref[...].T, preferred_element_type=jnp.float32)
    m = jnp.mean(h, axis=0, keepdims=True)
    v = jnp.mean(jnp.square(h), axis=0, keepdims=True) - jnp.square(m)
    h = jax.nn.relu((h - m) / jnp.sqrt(v + 1e-5))
    h2 = jnp.dot(h, w2_ref[...].T, preferred_element_type=jnp.float32)
    m2 = jnp.mean(h2, axis=0, keepdims=True)
    v2 = jnp.mean(jnp.square(h2), axis=0, keepdims=True) - jnp.square(m2)
    h2 = (h2 - m2) / jnp.sqrt(v2 + 1e-5)
    out_ref[...] = jax.nn.relu(h2 + agg)


def kernel(xyz, features, W_lse1, W_lse2, W_s1a, W_s1b, b_s1b, W_m1,
           W_s2a, W_s2b, b_s2b, W_m2, W_d1, W_d2):
    d = jnp.sum((xyz[:, :, None, :] - xyz[:, None, :, :]) ** 2, axis=-1)
    _, idx = jax.lax.top_k(-d, K)

    nb_xyz = _gather(xyz, idx)
    rel = nb_xyz - xyz[:, :, None, :]
    dist = jnp.sqrt(jnp.sum(rel ** 2, axis=-1, keepdims=True))
    nb_f = _gather(features, idx)
    pe = jnp.concatenate([rel, dist, nb_f], axis=-1)

    l1 = jax.nn.relu(_bn(jnp.einsum('bnkc,oc->bnko', pe, W_lse1)))
    a1 = _ap(l1, W_s1a, W_s1b, b_s1b, W_m1)
    l2 = jax.nn.relu(_bn(jnp.einsum('bnkc,oc->bnko', pe, W_lse2)))
    a2 = _ap(l2, W_s2a, W_s2b, b_s2b, W_m2)
    agg = jnp.concatenate([a1, a2], axis=-1)

    out = pl.pallas_call(
        _drb_body,
        out_shape=jax.ShapeDtypeStruct((B * N, OUT), jnp.float32),
    )(agg.reshape(B * N, OUT), W_d1, W_d2)
    return out.reshape(B, N, OUT)


# speedup vs baseline: 11.9348x; 11.9152x over previous
import functools

import jax
import jax.numpy as jnp
from jax import lax
from jax.experimental import pallas as pl
from jax.experimental.pallas import tpu as pltpu
from jax.experimental.pallas import tpu_sc as plsc

B, N, C = 4, 4096, 64
HALF, OUT, K = 64, 128, 16
W80 = 128
EPS = 1e-5
CNT_NK = float(B * N * K)
CNT_N = float(B * N)

RKNN = 256
PB = 128

SC_NC, SC_NS = 2, 16
SC_NW = SC_NC * SC_NS
TOT_ROWS = B * N * K
PER_W = TOT_ROWS // SC_NW
SC_CHUNK = 512


def _call(body, **kw):
    return pl.pallas_call(body, **kw)


def _knn_body(xyzp_ref, xyzt_ref, idx_ref):
    X = xyzp_ref[...]
    Y = xyzt_ref[0]
    d = ((X[:, 0:1] - Y[0:1, :]) ** 2
         + (X[:, 1:2] - Y[1:2, :]) ** 2
         + (X[:, 2:3] - Y[2:3, :]) ** 2)
    iota = lax.broadcasted_iota(jnp.int32, (RKNN, N), 1)
    iota16 = lax.broadcasted_iota(jnp.int32, (RKNN, K), 1)
    m = jnp.min(d, axis=1, keepdims=True)
    acc = jnp.zeros((RKNN, K), jnp.int32)
    for t in range(K):
        j = jnp.min(jnp.where(d == m, iota, N), axis=1, keepdims=True)
        acc = jnp.where(iota16 == t, j, acc)
        if t < K - 1:
            d = jnp.where(iota == j, jnp.inf, d)
            m = jnp.min(d, axis=1, keepdims=True)
    idx_ref[0] = acc


def _knn(xyzp80, xyzt):
    nb = N // RKNN
    return _call(
        _knn_body,
        grid=(B, nb),
        in_specs=[
            pl.BlockSpec((RKNN, W80), lambda b, r: (b * nb + r, 0)),
            pl.BlockSpec((1, 8, N), lambda b, r: (b, 0, 0)),
        ],
        out_specs=pl.BlockSpec((1, RKNN, K), lambda b, r: (b, r, 0)),
        out_shape=jax.ShapeDtypeStruct((B, N, K), jnp.int32),
    )(xyzp80, xyzt)


def _sc_gather(idx_flat, table):
    mesh = plsc.VectorSubcoreMesh(core_axis_name="c", subcore_axis_name="s")

    @functools.partial(
        pl.kernel,
        mesh=mesh,
        out_type=jax.ShapeDtypeStruct((TOT_ROWS, W80), jnp.float32),
        scratch_types=[
            pltpu.VMEM((SC_CHUNK,), jnp.int32),
            pltpu.VMEM((SC_CHUNK, W80), jnp.float32),
            pltpu.SemaphoreType.DMA,
        ],
    )
    def gather_k(idx_hbm, table_hbm, out_hbm, idx_v, rows_v, sem):
        wid = lax.axis_index("s") * SC_NC + lax.axis_index("c")
        base = wid * PER_W
        batch_base = (wid // (SC_NW // B)) * N

        def chunk(ci, _):
            off = base + ci * SC_CHUNK
            pltpu.sync_copy(idx_hbm.at[pl.ds(off, SC_CHUNK)], idx_v)

            def add_base(i, _):
                idx_v[pl.ds(i * 16, 16)] = idx_v[pl.ds(i * 16, 16)] + batch_base
                return 0

            lax.fori_loop(0, SC_CHUNK // 16, add_base, 0)
            pltpu.async_copy(table_hbm.at[idx_v], rows_v, sem).wait()
            pltpu.sync_copy(rows_v, out_hbm.at[pl.ds(off, SC_CHUNK)])
            return 0

        lax.fori_loop(0, PER_W // SC_CHUNK, chunk, 0)

    return gather_k(idx_flat, table)


def _fix_pe(pe_raw, ctr80):
    pe3 = pe_raw.reshape(PB, K, W80)
    sh = pe3 - ctr80[:, None, :]
    lane = lax.broadcasted_iota(jnp.int32, (1, 1, W80), 2)
    m3 = (lane < 3).astype(jnp.float32)
    dsq = jnp.sum(sh * sh * m3, axis=-1, keepdims=True)
    dist = jnp.sqrt(dsq)
    is3 = (lane == 3).astype(jnp.float32)
    pe = sh * (1.0 - is3) + dist * is3
    return pe.reshape(PB * K, W80)


def _mean_rstd(sums, sq, cnt):
    m = sums / cnt
    v = sq / cnt - m * m
    return m, lax.rsqrt(v + EPS)


def _stats_h_body(pe_ref, ctr_ref, w1_ref, w2_ref, out_ref):
    pe = _fix_pe(pe_ref[...], ctr_ref[...])
    h1 = jnp.dot(pe, w1_ref[...].T, preferred_element_type=jnp.float32)
    h2 = jnp.dot(pe, w2_ref[...].T, preferred_element_type=jnp.float32)
    part = jnp.stack([
        jnp.sum(h1, axis=0), jnp.sum(h1 * h1, axis=0),
        jnp.sum(h2, axis=0), jnp.sum(h2 * h2, axis=0),
    ])

    @pl.when(pl.program_id(0) == 0)
    def _():
        out_ref[...] = jnp.zeros_like(out_ref)

    out_ref[...] += part


def _stats_s_body(pe_ref, ctr_ref, w1_ref, w2_ref, hs_ref, wa1_ref, wa2_ref,
                  out_ref):
    pe = _fix_pe(pe_ref[...], ctr_ref[...])
    hs = hs_ref[...]
    m1, r1 = _mean_rstd(hs[0], hs[1], CNT_NK)
    m2, r2 = _mean_rstd(hs[2], hs[3], CNT_NK)
    h1 = jnp.dot(pe, w1_ref[...].T, preferred_element_type=jnp.float32)
    h2 = jnp.dot(pe, w2_ref[...].T, preferred_element_type=jnp.float32)
    x1 = jax.nn.relu((h1 - m1) * r1)
    x2 = jax.nn.relu((h2 - m2) * r2)
    s1 = jnp.dot(x1, wa1_ref[...].T, preferred_element_type=jnp.float32)
    s2 = jnp.dot(x2, wa2_ref[...].T, preferred_element_type=jnp.float32)
    part = jnp.stack([
        jnp.sum(s1, axis=0), jnp.sum(s1 * s1, axis=0),
        jnp.sum(s2, axis=0), jnp.sum(s2 * s2, axis=0),
    ])

    @pl.when(pl.program_id(0) == 0)
    def _():
        out_ref[...] = jnp.zeros_like(out_ref)

    out_ref[...] += part


def _pool_body(pe_ref, ctr_ref, w1_ref, w2_ref, hs_ref, wa1_ref, wa2_ref,
               ss_ref, wb1_ref, wb2_ref, bb_ref, wm1_ref, wm2_ref,
               g1_ref, g2_ref):
    pe = _fix_pe(pe_ref[...], ctr_ref[...])
    hs = hs_ref[...]
    ss = ss_ref[...]
    bb = bb_ref[...]

    def branch(w_ref, hoff, wa_ref, soff, wb_ref, boff, wm_ref):
        mh, rh = _mean_rstd(hs[hoff], hs[hoff + 1], CNT_NK)
        ms, rs = _mean_rstd(ss[soff], ss[soff + 1], CNT_NK)
        h = jnp.dot(pe, w_ref[...].T, preferred_element_type=jnp.float32)
        x = jax.nn.relu((h - mh) * rh)
        s = jnp.dot(x, wa_ref[...].T, preferred_element_type=jnp.float32)
        t = jax.nn.relu((s - ms) * rs)
        sc = jnp.sum(t * wb_ref[...], axis=1, keepdims=True) + bb[0, boff]
        sc3 = sc.reshape(PB, K, 1)
        sc3 = sc3 - jnp.max(sc3, axis=1, keepdims=True)
        e = jnp.exp(sc3)
        w = e / jnp.sum(e, axis=1, keepdims=True)
        f = jnp.sum(x.reshape(PB, K, HALF) * w, axis=1)
        return jnp.dot(f, wm_ref[...].T, preferred_element_type=jnp.float32)

    g1_ref[...] = branch(w1_ref, 0, wa1_ref, 0, wb1_ref, 0, wm1_ref)
    g2_ref[...] = branch(w2_ref, 2, wa2_ref, 2, wb2_ref, 1, wm2_ref)


def _decoder_body(g1_ref, g2_ref, wd1_ref, wd2_ref, out_ref):
    def norm_relu_inline(x):
        m = jnp.mean(x, axis=0, keepdims=True)
        v = jnp.mean(x * x, axis=0, keepdims=True) - m * m
        return jax.nn.relu((x - m) * lax.rsqrt(v + EPS))

    a1 = norm_relu_inline(g1_ref[...])
    a2 = norm_relu_inline(g2_ref[...])
    agg = jnp.concatenate([a1, a2], axis=1)
    d1 = jnp.dot(agg, wd1_ref[...].T, preferred_element_type=jnp.float32)
    u = norm_relu_inline(d1)
    d2 = jnp.dot(u, wd2_ref[...].T, preferred_element_type=jnp.float32)
    m = jnp.mean(d2, axis=0, keepdims=True)
    v = jnp.mean(d2 * d2, axis=0, keepdims=True) - m * m
    out_ref[...] = jax.nn.relu((d2 - m) * lax.rsqrt(v + EPS) + agg)


def kernel(xyz, features, W_lse1, W_lse2, W_s1a, W_s1b, b_s1b, W_m1,
           W_s2a, W_s2b, b_s2b, W_m2, W_d1, W_d2):
    f32 = jnp.float32
    xyzp80 = jnp.pad(xyz.reshape(B * N, 3), ((0, 0), (0, W80 - 3)))
    xyzt = jnp.pad(jnp.transpose(xyz, (0, 2, 1)), ((0, 0), (0, 5), (0, 0)))
    table = jnp.concatenate(
        [xyz, jnp.zeros((B, N, 1), f32), features,
         jnp.zeros((B, N, W80 - 4 - C), f32)], axis=-1).reshape(B * N, W80)
    w80_1 = jnp.pad(W_lse1, ((0, 0), (0, W80 - C - 4)))
    w80_2 = jnp.pad(W_lse2, ((0, 0), (0, W80 - C - 4)))
    bb = jnp.stack([b_s1b, b_s2b]).reshape(1, 2)

    idx = _knn(xyzp80, xyzt)
    idx_flat = idx.reshape(TOT_ROWS)

    pe_raw = _sc_gather(idx_flat, table)

    ng = (B * N) // PB
    row_spec = pl.BlockSpec((PB * K, W80), lambda i: (i, 0))
    ctr_spec = pl.BlockSpec((PB, W80), lambda i: (i, 0))
    full = lambda shape: pl.BlockSpec(shape, lambda i: tuple(0 for _ in shape))
    stats_shape = jax.ShapeDtypeStruct((4, HALF), f32)
    stats_spec = pl.BlockSpec((4, HALF), lambda i: (0, 0))

    stats_h = _call(
        _stats_h_body,
        grid=(ng,),
        in_specs=[row_spec, ctr_spec, full((HALF, W80)), full((HALF, W80))],
        out_specs=stats_spec,
        out_shape=stats_shape,
    )(pe_raw, xyzp80, w80_1, w80_2)

    stats_s = _call(
        _stats_s_body,
        grid=(ng,),
        in_specs=[row_spec, ctr_spec, full((HALF, W80)), full((HALF, W80)),
                  stats_spec, full((HALF, HALF)), full((HALF, HALF))],
        out_specs=stats_spec,
        out_shape=stats_shape,
    )(pe_raw, xyzp80, w80_1, w80_2, stats_h, W_s1a, W_s2a)

    g1, g2 = _call(
        _pool_body,
        grid=(ng,),
        in_specs=[row_spec, ctr_spec, full((HALF, W80)), full((HALF, W80)),
                  stats_spec, full((HALF, HALF)), full((HALF, HALF)),
                  stats_spec, full((1, HALF)), full((1, HALF)),
                  full((1, 2)), full((HALF, HALF)), full((HALF, HALF))],
        out_specs=[pl.BlockSpec((PB, HALF), lambda i: (i, 0))] * 2,
        out_shape=[jax.ShapeDtypeStruct((B * N, HALF), f32)] * 2,
    )(pe_raw, xyzp80, w80_1, w80_2, stats_h, W_s1a, W_s2a,
      stats_s, W_s1b, W_s2b, bb, W_m1, W_m2)

    out = _call(
        _decoder_body,
        out_shape=jax.ShapeDtypeStruct((B * N, OUT), f32),
    )(g1, g2, W_d1, W_d2)
    return out.reshape(B, N, OUT)


# speedup vs baseline: 13.1559x; 1.1023x over previous
import functools

import jax
import jax.numpy as jnp
from jax import lax
from jax.experimental import pallas as pl
from jax.experimental.pallas import tpu as pltpu
from jax.experimental.pallas import tpu_sc as plsc

B, N, C = 4, 4096, 64
HALF, OUT, K = 64, 128, 16
W80 = 128
EPS = 1e-5
CNT_NK = float(B * N * K)
CNT_N = float(B * N)

RKNN = 256
PB = 128

SC_NC, SC_NS = 2, 16
SC_NW = SC_NC * SC_NS
TOT_ROWS = B * N * K
PER_W = TOT_ROWS // SC_NW
SC_CHUNK = 512


def _call(body, **kw):
    return pl.pallas_call(body, **kw)


def _knn_body(xyzp_ref, xyzt_ref, idx_ref):
    X = xyzp_ref[...]
    Y = xyzt_ref[0]
    d = ((X[:, 0:1] - Y[0:1, :]) ** 2
         + (X[:, 1:2] - Y[1:2, :]) ** 2
         + (X[:, 2:3] - Y[2:3, :]) ** 2)
    fiota = lax.broadcasted_iota(jnp.int32, (RKNN, N), 1).astype(jnp.float32)
    iota16 = lax.broadcasted_iota(jnp.int32, (RKNN, K), 1)
    m = jnp.min(d, axis=1, keepdims=True)
    acc = jnp.zeros((RKNN, K), jnp.float32)
    for t in range(K):
        j = jnp.min(jnp.where(d == m, fiota, float(N)), axis=1, keepdims=True)
        acc = jnp.where(iota16 == t, j, acc)
        if t < K - 1:
            d = jnp.where(fiota == j, jnp.inf, d)
            m = jnp.min(d, axis=1, keepdims=True)
    idx_ref[0] = acc.astype(jnp.int32)


def _knn(xyzp80, xyzt):
    nb = N // RKNN
    return _call(
        _knn_body,
        grid=(B, nb),
        in_specs=[
            pl.BlockSpec((RKNN, W80), lambda b, r: (b * nb + r, 0)),
            pl.BlockSpec((1, 8, N), lambda b, r: (b, 0, 0)),
        ],
        out_specs=pl.BlockSpec((1, RKNN, K), lambda b, r: (b, r, 0)),
        out_shape=jax.ShapeDtypeStruct((B, N, K), jnp.int32),
    )(xyzp80, xyzt)


def _sc_gather(idx_flat, table):
    mesh = plsc.VectorSubcoreMesh(core_axis_name="c", subcore_axis_name="s")

    @functools.partial(
        pl.kernel,
        mesh=mesh,
        out_type=jax.ShapeDtypeStruct((TOT_ROWS, W80), jnp.float32),
        scratch_types=[
            pltpu.VMEM((SC_CHUNK,), jnp.int32),
            pltpu.VMEM((SC_CHUNK, W80), jnp.float32),
            pltpu.SemaphoreType.DMA,
        ],
    )
    def gather_k(idx_hbm, table_hbm, out_hbm, idx_v, rows_v, sem):
        wid = lax.axis_index("s") * SC_NC + lax.axis_index("c")
        base = wid * PER_W
        batch_base = (wid // (SC_NW // B)) * N

        def chunk(ci, _):
            off = base + ci * SC_CHUNK
            pltpu.sync_copy(idx_hbm.at[pl.ds(off, SC_CHUNK)], idx_v)

            def add_base(i, _):
                idx_v[pl.ds(i * 16, 16)] = idx_v[pl.ds(i * 16, 16)] + batch_base
                return 0

            lax.fori_loop(0, SC_CHUNK // 16, add_base, 0)
            pltpu.async_copy(table_hbm.at[idx_v], rows_v, sem).wait()
            pltpu.sync_copy(rows_v, out_hbm.at[pl.ds(off, SC_CHUNK)])
            return 0

        lax.fori_loop(0, PER_W // SC_CHUNK, chunk, 0)

    return gather_k(idx_flat, table)


def _fix_pe(pe_raw, ctr80):
    pe3 = pe_raw.reshape(PB, K, W80)
    sh = pe3 - ctr80[:, None, :]
    lane = lax.broadcasted_iota(jnp.int32, (1, 1, W80), 2)
    m3 = (lane < 3).astype(jnp.float32)
    dsq = jnp.sum(sh * sh * m3, axis=-1, keepdims=True)
    dist = jnp.sqrt(dsq)
    is3 = (lane == 3).astype(jnp.float32)
    pe = sh * (1.0 - is3) + dist * is3
    return pe.reshape(PB * K, W80)


def _mean_rstd(sums, sq, cnt):
    m = sums / cnt
    v = sq / cnt - m * m
    return m, lax.rsqrt(v + EPS)


def _stats_h_body(pe_ref, ctr_ref, w1_ref, w2_ref, out_ref):
    pe = _fix_pe(pe_ref[...], ctr_ref[...])
    h1 = jnp.dot(pe, w1_ref[...].T, preferred_element_type=jnp.float32)
    h2 = jnp.dot(pe, w2_ref[...].T, preferred_element_type=jnp.float32)
    part = jnp.stack([
        jnp.sum(h1, axis=0), jnp.sum(h1 * h1, axis=0),
        jnp.sum(h2, axis=0), jnp.sum(h2 * h2, axis=0),
    ])

    @pl.when(pl.program_id(0) == 0)
    def _():
        out_ref[...] = jnp.zeros_like(out_ref)

    out_ref[...] += part


def _stats_s_body(pe_ref, ctr_ref, w1_ref, w2_ref, hs_ref, wa1_ref, wa2_ref,
                  out_ref):
    pe = _fix_pe(pe_ref[...], ctr_ref[...])
    hs = hs_ref[...]
    m1, r1 = _mean_rstd(hs[0], hs[1], CNT_NK)
    m2, r2 = _mean_rstd(hs[2], hs[3], CNT_NK)
    h1 = jnp.dot(pe, w1_ref[...].T, preferred_element_type=jnp.float32)
    h2 = jnp.dot(pe, w2_ref[...].T, preferred_element_type=jnp.float32)
    x1 = jax.nn.relu((h1 - m1) * r1)
    x2 = jax.nn.relu((h2 - m2) * r2)
    s1 = jnp.dot(x1, wa1_ref[...].T, preferred_element_type=jnp.float32)
    s2 = jnp.dot(x2, wa2_ref[...].T, preferred_element_type=jnp.float32)
    part = jnp.stack([
        jnp.sum(s1, axis=0), jnp.sum(s1 * s1, axis=0),
        jnp.sum(s2, axis=0), jnp.sum(s2 * s2, axis=0),
    ])

    @pl.when(pl.program_id(0) == 0)
    def _():
        out_ref[...] = jnp.zeros_like(out_ref)

    out_ref[...] += part


def _pool_body(pe_ref, ctr_ref, w1_ref, w2_ref, hs_ref, wa1_ref, wa2_ref,
               ss_ref, wb1_ref, wb2_ref, bb_ref, wm1_ref, wm2_ref,
               g1_ref, g2_ref):
    pe = _fix_pe(pe_ref[...], ctr_ref[...])
    hs = hs_ref[...]
    ss = ss_ref[...]
    bb = bb_ref[...]

    seg = (lax.broadcasted_iota(jnp.int32, (PB, PB * K), 1) // K
           == lax.broadcasted_iota(jnp.int32, (PB, PB * K), 0)
           ).astype(jnp.float32)

    def branch(w_ref, hoff, wa_ref, soff, wb_ref, boff, wm_ref):
        mh, rh = _mean_rstd(hs[hoff], hs[hoff + 1], CNT_NK)
        ms, rs = _mean_rstd(ss[soff], ss[soff + 1], CNT_NK)
        h = jnp.dot(pe, w_ref[...].T, preferred_element_type=jnp.float32)
        x = jax.nn.relu((h - mh) * rh)
        s = jnp.dot(x, wa_ref[...].T, preferred_element_type=jnp.float32)
        t = jax.nn.relu((s - ms) * rs)
        sc = jnp.sum(t * wb_ref[...], axis=1) + bb[0, boff]
        scm = sc.reshape(PB, K)
        scm = scm - jnp.max(scm, axis=1, keepdims=True)
        e = jnp.exp(scm)
        w = e / jnp.sum(e, axis=1, keepdims=True)
        y = x * w.reshape(PB * K, 1)
        f = jnp.dot(seg, y, preferred_element_type=jnp.float32)
        return jnp.dot(f, wm_ref[...].T, preferred_element_type=jnp.float32)

    g1_ref[...] = branch(w1_ref, 0, wa1_ref, 0, wb1_ref, 0, wm1_ref)
    g2_ref[...] = branch(w2_ref, 2, wa2_ref, 2, wb2_ref, 1, wm2_ref)


def _decoder_body(g1_ref, g2_ref, wd1_ref, wd2_ref, out_ref):
    def norm_relu_inline(x):
        m = jnp.mean(x, axis=0, keepdims=True)
        v = jnp.mean(x * x, axis=0, keepdims=True) - m * m
        return jax.nn.relu((x - m) * lax.rsqrt(v + EPS))

    a1 = norm_relu_inline(g1_ref[...])
    a2 = norm_relu_inline(g2_ref[...])
    agg = jnp.concatenate([a1, a2], axis=1)
    d1 = jnp.dot(agg, wd1_ref[...].T, preferred_element_type=jnp.float32)
    u = norm_relu_inline(d1)
    d2 = jnp.dot(u, wd2_ref[...].T, preferred_element_type=jnp.float32)
    m = jnp.mean(d2, axis=0, keepdims=True)
    v = jnp.mean(d2 * d2, axis=0, keepdims=True) - m * m
    out_ref[...] = jax.nn.relu((d2 - m) * lax.rsqrt(v + EPS) + agg)


def kernel(xyz, features, W_lse1, W_lse2, W_s1a, W_s1b, b_s1b, W_m1,
           W_s2a, W_s2b, b_s2b, W_m2, W_d1, W_d2):
    f32 = jnp.float32
    xyzp80 = jnp.pad(xyz.reshape(B * N, 3), ((0, 0), (0, W80 - 3)))
    xyzt = jnp.pad(jnp.transpose(xyz, (0, 2, 1)), ((0, 0), (0, 5), (0, 0)))
    table = jnp.concatenate(
        [xyz, jnp.zeros((B, N, 1), f32), features,
         jnp.zeros((B, N, W80 - 4 - C), f32)], axis=-1).reshape(B * N, W80)
    w80_1 = jnp.pad(W_lse1, ((0, 0), (0, W80 - C - 4)))
    w80_2 = jnp.pad(W_lse2, ((0, 0), (0, W80 - C - 4)))
    bb = jnp.stack([b_s1b, b_s2b]).reshape(1, 2)

    idx = _knn(xyzp80, xyzt)
    idx_flat = idx.reshape(TOT_ROWS)

    pe_raw = _sc_gather(idx_flat, table)

    ng = (B * N) // PB
    row_spec = pl.BlockSpec((PB * K, W80), lambda i: (i, 0))
    ctr_spec = pl.BlockSpec((PB, W80), lambda i: (i, 0))
    full = lambda shape: pl.BlockSpec(shape, lambda i: tuple(0 for _ in shape))
    stats_shape = jax.ShapeDtypeStruct((4, HALF), f32)
    stats_spec = pl.BlockSpec((4, HALF), lambda i: (0, 0))

    stats_h = _call(
        _stats_h_body,
        grid=(ng,),
        in_specs=[row_spec, ctr_spec, full((HALF, W80)), full((HALF, W80))],
        out_specs=stats_spec,
        out_shape=stats_shape,
    )(pe_raw, xyzp80, w80_1, w80_2)

    stats_s = _call(
        _stats_s_body,
        grid=(ng,),
        in_specs=[row_spec, ctr_spec, full((HALF, W80)), full((HALF, W80)),
                  stats_spec, full((HALF, HALF)), full((HALF, HALF))],
        out_specs=stats_spec,
        out_shape=stats_shape,
    )(pe_raw, xyzp80, w80_1, w80_2, stats_h, W_s1a, W_s2a)

    g1, g2 = _call(
        _pool_body,
        grid=(ng,),
        in_specs=[row_spec, ctr_spec, full((HALF, W80)), full((HALF, W80)),
                  stats_spec, full((HALF, HALF)), full((HALF, HALF)),
                  stats_spec, full((1, HALF)), full((1, HALF)),
                  full((1, 2)), full((HALF, HALF)), full((HALF, HALF))],
        out_specs=[pl.BlockSpec((PB, HALF), lambda i: (i, 0))] * 2,
        out_shape=[jax.ShapeDtypeStruct((B * N, HALF), f32)] * 2,
    )(pe_raw, xyzp80, w80_1, w80_2, stats_h, W_s1a, W_s2a,
      stats_s, W_s1b, W_s2b, bb, W_m1, W_m2)

    out = _call(
        _decoder_body,
        out_shape=jax.ShapeDtypeStruct((B * N, OUT), f32),
    )(g1, g2, W_d1, W_d2)
    return out.reshape(B, N, OUT)
